# Initial kernel scaffold; baseline (speedup 1.0000x reference)
import jax
import jax.numpy as jnp
from jax.experimental import pallas as pl


def kernel(x, edge_index, W1, b1, bn_gamma, bn_beta, bn_mean, bn_var, W2, b2):
    raise NotImplementedError("write your pallas kernel here")


# speedup vs baseline: 12.3657x; 12.3657x over previous
import functools

import jax
import jax.numpy as jnp
from jax import lax
from jax.experimental import pallas as pl
from jax.experimental.pallas import tpu as pltpu
from jax.experimental.pallas import tpu_sc as plsc

N = 10000
F_IN = 128
H = 128
C = 40
E = 320000
EPS = 1e-5

NC = 2
NS = 16
NW = NC * NS
CH = 128
NCH = 80
PE = NW * NCH * CH
NP = 10240
STRIPE = NP // NS
KS = STRIPE // CH
G = 4

_mesh = plsc.VectorSubcoreMesh(
    core_axis_name="c", subcore_axis_name="s", num_cores=NC, num_subcores=NS)


def _make_deg_kernel():
    @functools.partial(
        pl.kernel,
        out_type=jax.ShapeDtypeStruct((NC * NP,), jnp.float32),
        mesh=_mesh,
        compiler_params=pltpu.CompilerParams(use_tc_tiling_on_sc=False),
        scratch_types=[
            pltpu.VMEM((NCH, CH), jnp.int32),
            pltpu.VMEM((CH,), jnp.float32),
            pltpu.VMEM((STRIPE,), jnp.float32),
            pltpu.VMEM_SHARED((NP,), jnp.float32),
            pltpu.SemaphoreType.DMA,
            pltpu.SemaphoreType.DMA,
            pltpu.SemaphoreType.DMA,
            pltpu.SemaphoreType.DMA,
        ],
    )
    def deg_kernel(dst3, out, dst_idx, ones, zbuf, acc, s0, s1, s2, s3):
        c = lax.axis_index("c")
        s = lax.axis_index("s")
        wid = c * NS + s
        pltpu.sync_copy(dst3.at[wid], dst_idx)
        for i in range(CH // 16):
            ones[pl.ds(i * 16, 16)] = jnp.ones((16,), jnp.float32)
        for i in range(STRIPE // 16):
            zbuf[pl.ds(i * 16, 16)] = jnp.zeros((16,), jnp.float32)
        rs = s * STRIPE
        pltpu.sync_copy(zbuf, acc.at[pl.ds(rs, STRIPE)])
        plsc.subcore_barrier()

        sems = [s0, s1, s2, s3]

        def group(g, _):
            descs = [
                pltpu.async_copy(ones, acc.at[dst_idx.at[g * G + b]], sems[b],
                                 add=True)
                for b in range(G)
            ]
            for d in descs:
                d.wait()
            return 0

        lax.fori_loop(0, NCH // G, group, 0)
        plsc.subcore_barrier()
        pltpu.sync_copy(acc.at[pl.ds(rs, STRIPE)], zbuf)
        pltpu.sync_copy(zbuf, out.at[pl.ds(c * NP + rs, STRIPE)])

    return deg_kernel


def _make_agg_kernel(d_feat, n_parts):
    @functools.partial(
        pl.kernel,
        out_type=[jax.ShapeDtypeStruct((NC, NP, d_feat), jnp.float32)
                  for _ in range(n_parts)],
        mesh=_mesh,
        compiler_params=pltpu.CompilerParams(use_tc_tiling_on_sc=False),
        scratch_types=[
            pltpu.VMEM((NCH, CH), jnp.int32),
            pltpu.VMEM((NCH, CH), jnp.int32),
            pltpu.VMEM((CH, d_feat), jnp.float32),
            pltpu.VMEM((CH, d_feat), jnp.float32),
            pltpu.VMEM((CH, d_feat), jnp.float32),
            pltpu.VMEM((CH, d_feat), jnp.float32),
            pltpu.VMEM_SHARED((NP, d_feat), jnp.float32),
            pltpu.SemaphoreType.DMA,
            pltpu.SemaphoreType.DMA,
            pltpu.SemaphoreType.DMA,
            pltpu.SemaphoreType.DMA,
        ],
    )
    def agg_kernel(*refs):
        ys = refs[:n_parts]
        zeros2 = refs[n_parts]
        src3 = refs[n_parts + 1]
        dst3 = refs[n_parts + 2]
        outs = refs[n_parts + 3:2 * n_parts + 3]
        (src_idx, dst_idx, r0, r1, r2, r3, acc,
         s0, s1, s2, s3) = refs[2 * n_parts + 3:]
        c = lax.axis_index("c")
        s = lax.axis_index("s")
        wid = c * NS + s
        pltpu.sync_copy(src3.at[wid], src_idx)
        pltpu.sync_copy(dst3.at[wid], dst_idx)
        rs = s * STRIPE

        bufs = [r0, r1, r2, r3]
        sems = [s0, s1, s2, s3]

        for y, out in zip(ys, outs):
            for k in range(KS):
                off = rs + k * CH

                @pl.when(c == 0)
                def _():
                    pltpu.sync_copy(y.at[pl.ds(off, CH)], r0)
                    pltpu.sync_copy(r0, acc.at[pl.ds(off, CH)])

                @pl.when(c != 0)
                def _():
                    pltpu.sync_copy(zeros2.at[pl.ds(off, CH)], r0)
                    pltpu.sync_copy(r0, acc.at[pl.ds(off, CH)])

            plsc.subcore_barrier()

            def group(g, _):
                descs = [
                    pltpu.async_copy(y.at[src_idx.at[g * G + b]], bufs[b],
                                     sems[b])
                    for b in range(G)
                ]
                for b in range(G):
                    descs[b].wait()
                    pltpu.sync_copy(bufs[b], acc.at[dst_idx.at[g * G + b]],
                                    add=True)
                return 0

            lax.fori_loop(0, NCH // G, group, 0)
            plsc.subcore_barrier()
            for k in range(KS):
                off = rs + k * CH
                pltpu.sync_copy(acc.at[pl.ds(off, CH)], r0)
                pltpu.sync_copy(r0, out.at[c, pl.ds(off, CH)])
            plsc.subcore_barrier()

    return agg_kernel


HH = H // 2
_deg_kernel = _make_deg_kernel()
_agg_h = _make_agg_kernel(HH, 2)
_agg_c = _make_agg_kernel(C, 1)

BR = 1280


def _y1_body(x_ref, w_ref, d0_ref, d1_ref, y_ref, dinv_ref):
    deg = d0_ref[...] + d1_ref[...] + 1.0
    dinv = lax.rsqrt(deg)
    xw = jnp.dot(x_ref[...], w_ref[...], preferred_element_type=jnp.float32)
    y_ref[...] = xw * dinv
    dinv_ref[...] = dinv


def _tc_y1(xp, w1, d0, d1):
    grid = (NP // BR,)
    return pl.pallas_call(
        _y1_body,
        grid=grid,
        in_specs=[
            pl.BlockSpec((BR, F_IN), lambda i: (i, 0)),
            pl.BlockSpec((F_IN, H), lambda i: (0, 0)),
            pl.BlockSpec((BR, 1), lambda i: (i, 0)),
            pl.BlockSpec((BR, 1), lambda i: (i, 0)),
        ],
        out_specs=[
            pl.BlockSpec((BR, H), lambda i: (i, 0)),
            pl.BlockSpec((BR, 1), lambda i: (i, 0)),
        ],
        out_shape=[
            jax.ShapeDtypeStruct((NP, H), jnp.float32),
            jax.ShapeDtypeStruct((NP, 1), jnp.float32),
        ],
    )(xp, w1, d0, d1)


def _mid_body(pa_ref, pb_ref, dinv_ref, g_ref, bt_ref, mn_ref, vr_ref,
              b1_ref, w2_ref, y2_ref):
    sc = g_ref[...] * lax.rsqrt(vr_ref[...] + EPS)
    t = (b1_ref[...] - mn_ref[...]) * sc + bt_ref[...]
    z = jnp.concatenate(
        [pa_ref[0] + pa_ref[1], pb_ref[0] + pb_ref[1]], axis=1)
    z = z * dinv_ref[...]
    h = jnp.maximum(z * sc + t, 0.0)
    y2_ref[...] = jnp.dot(h, w2_ref[...],
                          preferred_element_type=jnp.float32) * dinv_ref[...]


def _tc_mid(pa, pb, dinv, g, bt, mn, vr, b1, w2):
    grid = (NP // BR,)
    vspec = pl.BlockSpec((1, H), lambda i: (0, 0))
    return pl.pallas_call(
        _mid_body,
        grid=grid,
        in_specs=[
            pl.BlockSpec((NC, BR, HH), lambda i: (0, i, 0)),
            pl.BlockSpec((NC, BR, HH), lambda i: (0, i, 0)),
            pl.BlockSpec((BR, 1), lambda i: (i, 0)),
            vspec, vspec, vspec, vspec, vspec,
            pl.BlockSpec((H, C), lambda i: (0, 0)),
        ],
        out_specs=pl.BlockSpec((BR, C), lambda i: (i, 0)),
        out_shape=jax.ShapeDtypeStruct((NP, C), jnp.float32),
    )(pa, pb, dinv, g, bt, mn, vr, b1, w2)


def _out_body(q_ref, dinv_ref, b2_ref, o_ref):
    o_ref[...] = (q_ref[0] + q_ref[1]) * dinv_ref[...] + b2_ref[...]


def _tc_out(q, dinv, b2):
    grid = (NP // BR,)
    return pl.pallas_call(
        _out_body,
        grid=grid,
        in_specs=[
            pl.BlockSpec((NC, BR, C), lambda i: (0, i, 0)),
            pl.BlockSpec((BR, 1), lambda i: (i, 0)),
            pl.BlockSpec((1, C), lambda i: (0, 0)),
        ],
        out_specs=pl.BlockSpec((BR, C), lambda i: (i, 0)),
        out_shape=jax.ShapeDtypeStruct((NP, C), jnp.float32),
    )(q, dinv, b2)


def kernel(x, edge_index, W1, b1, bn_gamma, bn_beta, bn_mean, bn_var, W2, b2):
    xp = jnp.pad(x, ((0, NP - N), (0, 0)))
    pad = PE - E
    src = jnp.concatenate([edge_index[0], jnp.zeros((pad,), jnp.int32)])
    dst = jnp.concatenate([edge_index[1],
                           jnp.full((pad,), N, dtype=jnp.int32)])
    src3 = src.reshape(NW, NCH, CH)
    dst3 = dst.reshape(NW, NCH, CH)
    zeros_hh = jnp.zeros((NP, HH), jnp.float32)
    zeros_c = jnp.zeros((NP, C), jnp.float32)

    degs = _deg_kernel(dst3).reshape(NC, NP)
    y1, dinv = _tc_y1(xp, W1, degs[0][:, None], degs[1][:, None])
    ya = y1[:, :HH]
    yb = y1[:, HH:]
    pa, pb = _agg_h(ya, yb, zeros_hh, src3, dst3)
    y2 = _tc_mid(pa, pb, dinv,
                 bn_gamma.reshape(1, H), bn_beta.reshape(1, H),
                 bn_mean.reshape(1, H), bn_var.reshape(1, H),
                 b1.reshape(1, H), W2)
    (q,) = _agg_c(y2, zeros_c, src3, dst3)
    res = _tc_out(q, dinv, b2.reshape(1, C))
    return res[:N]
